# Initial kernel scaffold; baseline (speedup 1.0000x reference)
import jax
import jax.numpy as jnp
from jax.experimental import pallas as pl


def kernel(concat_embeddings, concat_type, pos_table, tok_table, ln_gamma, ln_beta):
    raise NotImplementedError("write your pallas kernel here")


# speedup vs baseline: 2.4387x; 2.4387x over previous
import functools

import jax
import jax.numpy as jnp
from jax import lax
from jax.experimental import pallas as pl
from jax.experimental.pallas import tpu as pltpu
from jax.experimental.pallas import tpu_sc as plsc

B, S, H = 64, 1024, 1024
EPS = 1e-12

_NC = 2
_NS = 16
_NW = _NC * _NS
_ROWS = B * S
_RPW = _ROWS // _NW
_CH = 64
_NCHUNK = _RPW // _CH


def _sc_gather(table, idx_flat):
    mesh = plsc.VectorSubcoreMesh(core_axis_name="c", subcore_axis_name="s")

    @functools.partial(
        pl.kernel,
        out_type=jax.ShapeDtypeStruct((_ROWS, H), jnp.float32),
        mesh=mesh,
        scratch_types=[
            pltpu.VMEM((_RPW,), jnp.int32),
            pltpu.VMEM((_CH, H), jnp.float32),
            pltpu.SemaphoreType.DMA,
        ],
    )
    def k(table_hbm, idx_hbm, out_hbm, idx_v, rows_v, sem):
        wid = lax.axis_index("s") * _NC + lax.axis_index("c")
        base = wid * _RPW
        pltpu.sync_copy(idx_hbm.at[pl.ds(base, _RPW)], idx_v)

        def body(c, carry):
            off = c * _CH
            pltpu.async_copy(
                table_hbm.at[idx_v.at[pl.ds(off, _CH)]], rows_v, sem
            ).wait()
            pltpu.sync_copy(rows_v, out_hbm.at[pl.ds(base + off, _CH)])
            return carry

        lax.fori_loop(0, _NCHUNK, body, 0)

    return k(table, idx_flat)


_R = 512


def _tc_add_ln(concat, tok, pos, gamma, beta):
    grid = (S // _R, B)

    def body(x_ref, t_ref, p_ref, g_ref, b_ref, o_ref):
        e = x_ref[...] + t_ref[...] + p_ref[...][None]
        mean = jnp.mean(e, axis=-1, keepdims=True)
        var = jnp.mean(jnp.square(e - mean), axis=-1, keepdims=True)
        xhat = (e - mean) * lax.rsqrt(var + EPS)
        o_ref[...] = xhat * g_ref[...] + b_ref[...]

    return pl.pallas_call(
        body,
        grid=grid,
        in_specs=[
            pl.BlockSpec((1, _R, H), lambda j, b: (b, j, 0)),
            pl.BlockSpec((1, _R, H), lambda j, b: (b, j, 0)),
            pl.BlockSpec((_R, H), lambda j, b: (j, 0)),
            pl.BlockSpec((1, H), lambda j, b: (0, 0)),
            pl.BlockSpec((1, H), lambda j, b: (0, 0)),
        ],
        out_specs=pl.BlockSpec((1, _R, H), lambda j, b: (b, j, 0)),
        out_shape=jax.ShapeDtypeStruct((B, S, H), jnp.float32),
    )(concat, tok, pos, gamma, beta)


def kernel(concat_embeddings, concat_type, pos_table, tok_table, ln_gamma, ln_beta):
    idx_flat = concat_type.reshape(-1).astype(jnp.int32)
    tok = _sc_gather(tok_table, idx_flat).reshape(B, S, H)
    return _tc_add_ln(
        concat_embeddings,
        tok,
        pos_table,
        ln_gamma.reshape(1, H),
        ln_beta.reshape(1, H),
    )


# speedup vs baseline: 2.4995x; 1.0249x over previous
import functools

import jax
import jax.numpy as jnp
from jax import lax
from jax.experimental import pallas as pl
from jax.experimental.pallas import tpu as pltpu
from jax.experimental.pallas import tpu_sc as plsc

B, S, H = 64, 1024, 1024
EPS = 1e-12

_NC = 2
_NS = 16
_NW = _NC * _NS
_ROWS = B * S
_RPW = _ROWS // _NW
_CH = 32
_NCHUNK = _RPW // _CH


def _sc_gather(table, idx_flat):
    mesh = plsc.VectorSubcoreMesh(core_axis_name="c", subcore_axis_name="s")

    @functools.partial(
        pl.kernel,
        out_type=jax.ShapeDtypeStruct((_ROWS, H), jnp.float32),
        mesh=mesh,
        scratch_types=[
            pltpu.VMEM((_RPW,), jnp.int32),
            pltpu.VMEM((_CH, H), jnp.float32),
            pltpu.VMEM((_CH, H), jnp.float32),
            pltpu.SemaphoreType.DMA,
            pltpu.SemaphoreType.DMA,
        ],
    )
    def k(table_hbm, idx_hbm, out_hbm, idx_v, buf0, buf1, sem0, sem1):
        wid = lax.axis_index("s") * _NC + lax.axis_index("c")
        base = wid * _RPW
        pltpu.sync_copy(idx_hbm.at[pl.ds(base, _RPW)], idx_v)
        pltpu.async_copy(table_hbm.at[idx_v.at[pl.ds(0, _CH)]], buf0, sem0)

        def step(c, cur, cur_sem, nxt, nxt_sem):
            @pl.when(c + 1 < _NCHUNK)
            def _():
                pltpu.async_copy(
                    table_hbm.at[idx_v.at[pl.ds((c + 1) * _CH, _CH)]],
                    nxt, nxt_sem,
                )
            pltpu.make_async_copy(
                table_hbm.at[idx_v.at[pl.ds(c * _CH, _CH)]], cur, cur_sem
            ).wait()
            pltpu.sync_copy(cur, out_hbm.at[pl.ds(base + c * _CH, _CH)])

        def body(c, carry):
            @pl.when(c % 2 == 0)
            def _():
                step(c, buf0, sem0, buf1, sem1)

            @pl.when(c % 2 == 1)
            def _():
                step(c, buf1, sem1, buf0, sem0)

            return carry

        lax.fori_loop(0, _NCHUNK, body, 0)

    return k(table, idx_flat)


_R = 512


def _tc_add_ln(concat, tok, pos, gamma, beta):
    grid = (S // _R, B)

    def body(x_ref, t_ref, p_ref, g_ref, b_ref, o_ref):
        e = x_ref[...] + t_ref[...] + p_ref[...][None]
        mean = jnp.mean(e, axis=-1, keepdims=True)
        var = jnp.mean(jnp.square(e - mean), axis=-1, keepdims=True)
        xhat = (e - mean) * lax.rsqrt(var + EPS)
        o_ref[...] = xhat * g_ref[...] + b_ref[...]

    return pl.pallas_call(
        body,
        grid=grid,
        in_specs=[
            pl.BlockSpec((1, _R, H), lambda j, b: (b, j, 0)),
            pl.BlockSpec((1, _R, H), lambda j, b: (b, j, 0)),
            pl.BlockSpec((_R, H), lambda j, b: (j, 0)),
            pl.BlockSpec((1, H), lambda j, b: (0, 0)),
            pl.BlockSpec((1, H), lambda j, b: (0, 0)),
        ],
        out_specs=pl.BlockSpec((1, _R, H), lambda j, b: (b, j, 0)),
        out_shape=jax.ShapeDtypeStruct((B, S, H), jnp.float32),
    )(concat, tok, pos, gamma, beta)


def kernel(concat_embeddings, concat_type, pos_table, tok_table, ln_gamma, ln_beta):
    idx_flat = concat_type.reshape(-1).astype(jnp.int32)
    tok = _sc_gather(tok_table, idx_flat).reshape(B, S, H)
    return _tc_add_ln(
        concat_embeddings,
        tok,
        pos_table,
        ln_gamma.reshape(1, H),
        ln_beta.reshape(1, H),
    )
